# baseline (speedup 1.0000x reference)
import functools

import jax
import jax.numpy as jnp
from jax import lax
from jax.experimental import pallas as pl
from jax.experimental.pallas import tpu as pltpu
from jax.experimental.pallas import tpu_sc as plsc

E = 1600000
KEEP = 800000
D = 32

_NC = 2
_NS = 16
_NW = _NC * _NS
_BPW = KEEP // _NW


def _leaky(x):
    return jnp.where(x >= 0, x, 0.2 * x)


def _bn(x, g, b):
    m = jnp.mean(x, axis=0, keepdims=True)
    v = jnp.var(x, axis=0, keepdims=True)
    return (x - m) / jnp.sqrt(v + 1e-5) * g + b


_mesh = plsc.VectorSubcoreMesh(core_axis_name="c", subcore_axis_name="s")


@functools.partial(
    pl.kernel,
    mesh=_mesh,
    out_type=[
        jax.ShapeDtypeStruct((KEEP,), jnp.float32),
        jax.ShapeDtypeStruct((KEEP,), jnp.int32),
        jax.ShapeDtypeStruct((KEEP,), jnp.int32),
    ],
    scratch_types=[
        pltpu.VMEM((_BPW,), jnp.int32),
        pltpu.VMEM((_BPW,), jnp.float32),
        pltpu.VMEM((_BPW,), jnp.int32),
        pltpu.VMEM((_BPW,), jnp.int32),
        pltpu.SemaphoreType.DMA,
    ],
)
def _sc_gather3(vals_hbm, r0_hbm, r1_hbm, locs_hbm,
                ov_hbm, o0_hbm, o1_hbm,
                idx_v, vv, v0, v1, sem):
    wid = lax.axis_index("s") * _NC + lax.axis_index("c")
    base = wid * _BPW
    pltpu.sync_copy(locs_hbm.at[pl.ds(base, _BPW)], idx_v)
    pltpu.async_copy(vals_hbm.at[idx_v], vv, sem).wait()
    pltpu.async_copy(r0_hbm.at[idx_v], v0, sem).wait()
    pltpu.async_copy(r1_hbm.at[idx_v], v1, sem).wait()
    pltpu.sync_copy(vv, ov_hbm.at[pl.ds(base, _BPW)])
    pltpu.sync_copy(v0, o0_hbm.at[pl.ds(base, _BPW)])
    pltpu.sync_copy(v1, o1_hbm.at[pl.ds(base, _BPW)])


_EPW = E // _NW
_CH = 2000
_NCHUNK = _EPW // _CH


@functools.partial(
    pl.kernel,
    mesh=_mesh,
    out_type=[
        jax.ShapeDtypeStruct((E, D), jnp.float32),
        jax.ShapeDtypeStruct((E, D), jnp.float32),
    ],
    scratch_types=[
        pltpu.VMEM((_CH,), jnp.int32),
        pltpu.VMEM((_CH, D), jnp.float32),
        pltpu.SemaphoreType.DMA,
    ],
    compiler_params=pltpu.CompilerParams(use_tc_tiling_on_sc=False, needs_layout_passes=False),
)
def _sc_gather_keys(tab_u_hbm, tab_i_hbm, idx_u_hbm, idx_i_hbm,
                    ou_hbm, oi_hbm, idx_v, rows_v, sem):
    wid = lax.axis_index("s") * _NC + lax.axis_index("c")
    base = wid * _EPW
    for tab, idxs, out in ((tab_u_hbm, idx_u_hbm, ou_hbm),
                           (tab_i_hbm, idx_i_hbm, oi_hbm)):
        for c in range(_NCHUNK):
            off = base + c * _CH
            pltpu.sync_copy(idxs.at[pl.ds(off, _CH)], idx_v)
            pltpu.async_copy(tab.at[idx_v], rows_v, sem).wait()
            pltpu.sync_copy(rows_v, out.at[pl.ds(off, _CH), :])


_LB = 1566
_SCK = 16 * _LB
_TPW = 2 * _SCK
_EPAD = _NW * _TPW
_NBIN = 256 * 2 * 16


def _make_hist_kernel(shift):
    @functools.partial(
        pl.kernel,
        mesh=_mesh,
        out_type=jax.ShapeDtypeStruct((_NW, _NBIN), jnp.int32),
        scratch_types=[
            pltpu.VMEM((_SCK,), jnp.int32),
            pltpu.VMEM((_NBIN,), jnp.int32),
            pltpu.SemaphoreType.DMA,
        ],
        compiler_params=pltpu.CompilerParams(use_tc_tiling_on_sc=False, needs_layout_passes=False),
    )
    def hist_k(keys_hbm, hist_hbm, keys_c, hist_v, sem):
        wid = lax.axis_index("s") * _NC + lax.axis_index("c")
        lanes = lax.iota(jnp.int32, 16)
        zeros = jnp.zeros((16,), jnp.int32)
        ones = jnp.ones((16,), jnp.int32)

        def zbody(j, _):
            hist_v[pl.ds(j * 16, 16)] = zeros
            return 0
        lax.fori_loop(0, _NBIN // 16, zbody, 0)

        for s in range(2):
            pltpu.sync_copy(
                keys_hbm.at[pl.ds(wid * _TPW + s * _SCK, _SCK)], keys_c)

            def body(i, _, s=s):
                addr = lanes * _LB + i
                k = plsc.load_gather(keys_c, [addr])
                d = lax.shift_right_logical(k, shift) & 255
                cidx = d * 32 + (s * 16) + lanes
                plsc.addupdate_scatter(hist_v, [cidx], ones)
                return 0
            lax.fori_loop(0, _LB, body, 0)
        pltpu.sync_copy(hist_v, hist_hbm.at[wid])
    return hist_k


def _make_perm_kernel(shift, write_keys):
    if write_keys:
        out_type = [jax.ShapeDtypeStruct((_EPAD,), jnp.int32)] * 2
    else:
        out_type = jax.ShapeDtypeStruct((_EPAD,), jnp.int32)

    @functools.partial(
        pl.kernel,
        mesh=_mesh,
        out_type=out_type,
        scratch_types=[
            pltpu.VMEM((_SCK,), jnp.int32),
            pltpu.VMEM((_SCK,), jnp.int32),
            pltpu.VMEM((_SCK,), jnp.int32),
            pltpu.VMEM((_NBIN,), jnp.int32),
            pltpu.SemaphoreType.DMA,
        ],
        compiler_params=pltpu.CompilerParams(use_tc_tiling_on_sc=False, needs_layout_passes=False),
    )
    def perm_k(keys_hbm, pay_hbm, bases_hbm, *rest):
        if write_keys:
            keys_out, pay_out, keys_c, pay_c, offs_c, bases_v, sem = rest
        else:
            pay_out, keys_c, pay_c, offs_c, bases_v, sem = rest
        wid = lax.axis_index("s") * _NC + lax.axis_index("c")
        lanes = lax.iota(jnp.int32, 16)
        pltpu.sync_copy(bases_hbm.at[wid], bases_v)
        for s in range(2):
            off0 = wid * _TPW + s * _SCK
            pltpu.sync_copy(keys_hbm.at[pl.ds(off0, _SCK)], keys_c)
            pltpu.sync_copy(pay_hbm.at[pl.ds(off0, _SCK)], pay_c)

            def body(i, _, s=s):
                addr = lanes * _LB + i
                k = plsc.load_gather(keys_c, [addr])
                d = lax.shift_right_logical(k, shift) & 255
                cidx = d * 32 + (s * 16) + lanes
                dst = plsc.load_gather(bases_v, [cidx])
                plsc.store_scatter(bases_v, [cidx], dst + 1)
                plsc.store_scatter(offs_c, [addr], dst)
                return 0
            lax.fori_loop(0, _LB, body, 0)
            if write_keys:
                pltpu.async_copy(keys_c, keys_out.at[offs_c], sem).wait()
            pltpu.async_copy(pay_c, pay_out.at[offs_c], sem).wait()
    return perm_k


_hist_ks = [_make_hist_kernel(8 * p) for p in range(4)]
_perm_ks = [_make_perm_kernel(8 * p, p < 3) for p in range(4)]


def _radix_bases(hist):
    h = hist.reshape(_NW, 256, 2, 16).transpose(1, 0, 2, 3).reshape(-1)
    c = jnp.concatenate([jnp.zeros((1,), jnp.int32), jnp.cumsum(h)[:-1]])
    return c.reshape(256, _NW, 2, 16).transpose(1, 0, 2, 3).reshape(_NW, _NBIN)


def _sc_topk_locs(scores):
    keys = jnp.bitwise_not(lax.bitcast_convert_type(scores, jnp.int32))
    keys = jnp.concatenate(
        [keys, jnp.full((_EPAD - E,), -1, jnp.int32)])
    pays = lax.iota(jnp.int32, _EPAD)
    for p in range(4):
        bases = _radix_bases(_hist_ks[p](keys))
        if p < 3:
            keys, pays = _perm_ks[p](keys, pays, bases)
        else:
            pays = _perm_ks[p](keys, pays, bases)
    return pays[:KEEP]


def kernel(trn_rows, trn_cols, edgeids, adj_vals, adj_idxs, ui_uKey, ui_iKey,
           ui_uHyper, ui_iHyper, Wm1, bm1, Wm2, bm2, Wl1, bl1, Wl2, bl2,
           g1, be1, g2, be2):
    uK = jnp.reshape(ui_uKey, (-1, D))
    iK = jnp.reshape(ui_iKey, (-1, D))
    usrKey, itmKey = _sc_gather_keys(uK, iK, trn_rows, trn_cols)

    def meta_map(hyper, keyv):
        hm = jnp.mean(hyper, axis=0, keepdims=True)
        W1 = jnp.reshape(hm @ Wm1 + bm1, (D, D))
        b1 = hm @ Wm2 + bm2
        return _leaky(keyv @ W1 + b1)

    ulat = meta_map(ui_uHyper, usrKey)
    ilat = meta_map(ui_iHyper, itmKey)
    lat = jnp.concatenate((ulat, ilat), axis=-1)
    lat = _leaky(_bn(lat @ Wl1 + bl1, g1, be1)) + ulat + ilat
    scores = jnp.reshape(jax.nn.sigmoid(_bn(lat @ Wl2 + bl2, g2, be2)), (-1,))
    topLocs = _sc_topk_locs(scores)

    nv, n0, n1 = _sc_gather3(adj_vals, adj_idxs[0], adj_idxs[1], topLocs)
    return (nv, jnp.stack((n0, n1)))


# speedup vs baseline: 3.4624x; 3.4624x over previous
import functools

import jax
import jax.numpy as jnp
from jax import lax
from jax.experimental import pallas as pl
from jax.experimental.pallas import tpu as pltpu
from jax.experimental.pallas import tpu_sc as plsc

E = 1600000
KEEP = 800000
D = 32

_NC = 2
_NS = 16
_NW = _NC * _NS
_BPW = KEEP // _NW


def _leaky(x):
    return jnp.where(x >= 0, x, 0.2 * x)


def _bn(x, g, b):
    m = jnp.mean(x, axis=0, keepdims=True)
    v = jnp.var(x, axis=0, keepdims=True)
    return (x - m) / jnp.sqrt(v + 1e-5) * g + b


_mesh = plsc.VectorSubcoreMesh(core_axis_name="c", subcore_axis_name="s")


@functools.partial(
    pl.kernel,
    mesh=_mesh,
    out_type=[
        jax.ShapeDtypeStruct((KEEP,), jnp.float32),
        jax.ShapeDtypeStruct((KEEP,), jnp.int32),
        jax.ShapeDtypeStruct((KEEP,), jnp.int32),
    ],
    scratch_types=[
        pltpu.VMEM((_BPW,), jnp.int32),
        pltpu.VMEM((_BPW,), jnp.float32),
        pltpu.VMEM((_BPW,), jnp.int32),
        pltpu.VMEM((_BPW,), jnp.int32),
        pltpu.SemaphoreType.DMA,
    ],
)
def _sc_gather3(vals_hbm, r0_hbm, r1_hbm, locs_hbm,
                ov_hbm, o0_hbm, o1_hbm,
                idx_v, vv, v0, v1, sem):
    wid = lax.axis_index("s") * _NC + lax.axis_index("c")
    base = wid * _BPW
    pltpu.sync_copy(locs_hbm.at[pl.ds(base, _BPW)], idx_v)
    pltpu.async_copy(vals_hbm.at[idx_v], vv, sem).wait()
    pltpu.async_copy(r0_hbm.at[idx_v], v0, sem).wait()
    pltpu.async_copy(r1_hbm.at[idx_v], v1, sem).wait()
    pltpu.sync_copy(vv, ov_hbm.at[pl.ds(base, _BPW)])
    pltpu.sync_copy(v0, o0_hbm.at[pl.ds(base, _BPW)])
    pltpu.sync_copy(v1, o1_hbm.at[pl.ds(base, _BPW)])


_EPW = E // _NW
_CH = 1000
_NCHUNK = _EPW // _CH


@functools.partial(
    pl.kernel,
    mesh=_mesh,
    out_type=[
        jax.ShapeDtypeStruct((E, D), jnp.float32),
        jax.ShapeDtypeStruct((E, D), jnp.float32),
    ],
    scratch_types=[
        pltpu.VMEM((2, _CH), jnp.int32),
        pltpu.VMEM((2, _CH, D), jnp.float32),
        pltpu.SemaphoreType.DMA,
        pltpu.SemaphoreType.DMA,
        pltpu.SemaphoreType.DMA,
    ],
    compiler_params=pltpu.CompilerParams(use_tc_tiling_on_sc=False, needs_layout_passes=False),
)
def _sc_gather_keys(tab_u_hbm, tab_i_hbm, idx_u_hbm, idx_i_hbm,
                    ou_hbm, oi_hbm, idx_v, rows_v, gsem, osem0, osem1):
    wid = lax.axis_index("s") * _NC + lax.axis_index("c")
    base = wid * _EPW
    osems = (osem0, osem1)
    plan = []
    for tab, idxs, out in ((tab_u_hbm, idx_u_hbm, ou_hbm),
                           (tab_i_hbm, idx_i_hbm, oi_hbm)):
        for c in range(_NCHUNK):
            off = base + c * _CH
            plan.append((tab, idxs.at[pl.ds(off, _CH)], out.at[pl.ds(off, _CH), :]))

    prev_out = [None, None]
    for n, (tab, idx_slice, out_slice) in enumerate(plan):
        b = n % 2
        if prev_out[b] is not None:
            prev_out[b].wait()
        pltpu.sync_copy(idx_slice, idx_v.at[b])
        pltpu.async_copy(tab.at[idx_v.at[b]], rows_v.at[b], gsem).wait()
        prev_out[b] = pltpu.async_copy(rows_v.at[b], out_slice, osems[b])
    for b in range(2):
        if prev_out[b] is not None:
            prev_out[b].wait()


_LB = 1566
_SCK = 16 * _LB
_TPW = 2 * _SCK
_EPAD = _NW * _TPW
_NBIN = 256 * 2 * 16


def _make_hist_kernel(shift):
    @functools.partial(
        pl.kernel,
        mesh=_mesh,
        out_type=jax.ShapeDtypeStruct((_NW, _NBIN), jnp.int32),
        scratch_types=[
            pltpu.VMEM((_SCK,), jnp.int32),
            pltpu.VMEM((_NBIN,), jnp.int32),
            pltpu.SemaphoreType.DMA,
        ],
        compiler_params=pltpu.CompilerParams(use_tc_tiling_on_sc=False, needs_layout_passes=False),
    )
    def hist_k(keys_hbm, hist_hbm, keys_c, hist_v, sem):
        wid = lax.axis_index("s") * _NC + lax.axis_index("c")
        lanes = lax.iota(jnp.int32, 16)
        zeros = jnp.zeros((16,), jnp.int32)
        ones = jnp.ones((16,), jnp.int32)

        def zbody(j, _):
            hist_v[pl.ds(j * 16, 16)] = zeros
            return 0
        lax.fori_loop(0, _NBIN // 16, zbody, 0)

        for s in range(2):
            pltpu.sync_copy(
                keys_hbm.at[pl.ds(wid * _TPW + s * _SCK, _SCK)], keys_c)

            def body(i, _, s=s):
                addr = lanes * _LB + i
                k = plsc.load_gather(keys_c, [addr])
                d = lax.shift_right_logical(k, shift) & 255
                cidx = d * 32 + (s * 16) + lanes
                plsc.addupdate_scatter(hist_v, [cidx], ones)
                return 0
            lax.fori_loop(0, _LB, body, 0)
        pltpu.sync_copy(hist_v, hist_hbm.at[wid])
    return hist_k


def _make_perm_kernel(shift, write_keys):
    if write_keys:
        out_type = [jax.ShapeDtypeStruct((_EPAD,), jnp.int32)] * 2
    else:
        out_type = jax.ShapeDtypeStruct((_EPAD,), jnp.int32)

    @functools.partial(
        pl.kernel,
        mesh=_mesh,
        out_type=out_type,
        scratch_types=[
            pltpu.VMEM((_SCK,), jnp.int32),
            pltpu.VMEM((_SCK,), jnp.int32),
            pltpu.VMEM((_SCK,), jnp.int32),
            pltpu.VMEM((_NBIN,), jnp.int32),
            pltpu.SemaphoreType.DMA,
        ],
        compiler_params=pltpu.CompilerParams(use_tc_tiling_on_sc=False, needs_layout_passes=False),
    )
    def perm_k(keys_hbm, pay_hbm, bases_hbm, *rest):
        if write_keys:
            keys_out, pay_out, keys_c, pay_c, offs_c, bases_v, sem = rest
        else:
            pay_out, keys_c, pay_c, offs_c, bases_v, sem = rest
        wid = lax.axis_index("s") * _NC + lax.axis_index("c")
        lanes = lax.iota(jnp.int32, 16)
        pltpu.sync_copy(bases_hbm.at[wid], bases_v)
        for s in range(2):
            off0 = wid * _TPW + s * _SCK
            pltpu.sync_copy(keys_hbm.at[pl.ds(off0, _SCK)], keys_c)
            pltpu.sync_copy(pay_hbm.at[pl.ds(off0, _SCK)], pay_c)

            def body(i, _, s=s):
                addr = lanes * _LB + i
                k = plsc.load_gather(keys_c, [addr])
                d = lax.shift_right_logical(k, shift) & 255
                cidx = d * 32 + (s * 16) + lanes
                dst = plsc.load_gather(bases_v, [cidx])
                plsc.store_scatter(bases_v, [cidx], dst + 1)
                plsc.store_scatter(offs_c, [addr], dst)
                return 0
            lax.fori_loop(0, _LB, body, 0)
            if write_keys:
                pltpu.async_copy(keys_c, keys_out.at[offs_c], sem).wait()
            pltpu.async_copy(pay_c, pay_out.at[offs_c], sem).wait()
    return perm_k


_hist_ks = [_make_hist_kernel(8 * p) for p in range(4)]
_perm_ks = [_make_perm_kernel(8 * p, p < 3) for p in range(4)]


def _radix_bases(hist):
    h = hist.reshape(_NW, 256, 2, 16).transpose(1, 0, 2, 3).reshape(-1)
    c = jnp.concatenate([jnp.zeros((1,), jnp.int32), jnp.cumsum(h)[:-1]])
    return c.reshape(256, _NW, 2, 16).transpose(1, 0, 2, 3).reshape(_NW, _NBIN)


def _sc_topk_locs(scores):
    keys = jnp.bitwise_not(lax.bitcast_convert_type(scores, jnp.int32))
    keys = jnp.concatenate(
        [keys, jnp.full((_EPAD - E,), -1, jnp.int32)])
    pays = lax.iota(jnp.int32, _EPAD)
    for p in range(4):
        bases = _radix_bases(_hist_ks[p](keys))
        if p < 3:
            keys, pays = _perm_ks[p](keys, pays, bases)
        else:
            pays = _perm_ks[p](keys, pays, bases)
    return pays[:KEEP]


def kernel(trn_rows, trn_cols, edgeids, adj_vals, adj_idxs, ui_uKey, ui_iKey,
           ui_uHyper, ui_iHyper, Wm1, bm1, Wm2, bm2, Wl1, bl1, Wl2, bl2,
           g1, be1, g2, be2):
    uK = jnp.reshape(ui_uKey, (-1, D))
    iK = jnp.reshape(ui_iKey, (-1, D))
    usrKey, itmKey = _sc_gather_keys(uK, iK, trn_rows, trn_cols)

    def meta_map(hyper, keyv):
        hm = jnp.mean(hyper, axis=0, keepdims=True)
        W1 = jnp.reshape(hm @ Wm1 + bm1, (D, D))
        b1 = hm @ Wm2 + bm2
        return _leaky(keyv @ W1 + b1)

    ulat = meta_map(ui_uHyper, usrKey)
    ilat = meta_map(ui_iHyper, itmKey)
    lat = jnp.concatenate((ulat, ilat), axis=-1)
    lat = _leaky(_bn(lat @ Wl1 + bl1, g1, be1)) + ulat + ilat
    scores = jnp.reshape(jax.nn.sigmoid(_bn(lat @ Wl2 + bl2, g2, be2)), (-1,))
    _, topLocs = lax.top_k(scores, KEEP)

    nv, n0, n1 = _sc_gather3(adj_vals, adj_idxs[0], adj_idxs[1], topLocs)
    return (nv, jnp.stack((n0, n1)))
